# baseline (speedup 1.0000x reference)
import jax
import jax.numpy as jnp
from jax.experimental import pallas as pl
from jax.experimental.pallas import tpu as pltpu

N_TOKENS = 16384
HIDDEN_DIM = 2048
NUM_EXPERTS = 64
TILE = 1024


def _router_kernel(x_ref, wt_ref, b_ref, o_ref):
    x = x_ref[...].astype(jnp.bfloat16)
    w = wt_ref[...].astype(jnp.bfloat16)
    logits = jax.lax.dot_general(
        x, w, (((1,), (0,)), ((), ())),
        preferred_element_type=jnp.float32,
    ) + b_ref[...]
    m = jnp.max(logits, axis=-1, keepdims=True)
    e = jnp.exp(logits - m)
    o_ref[...] = e / jnp.sum(e, axis=-1, keepdims=True)


def kernel(x, W, b):
    wt = W.T
    b2 = b.reshape(1, NUM_EXPERTS)
    return pl.pallas_call(
        _router_kernel,
        grid=(N_TOKENS // TILE,),
        in_specs=[
            pl.BlockSpec((TILE, HIDDEN_DIM), lambda i: (i, 0)),
            pl.BlockSpec((HIDDEN_DIM, NUM_EXPERTS), lambda i: (0, 0)),
            pl.BlockSpec((1, NUM_EXPERTS), lambda i: (0, 0)),
        ],
        out_specs=pl.BlockSpec((TILE, NUM_EXPERTS), lambda i: (i, 0)),
        out_shape=jax.ShapeDtypeStruct((N_TOKENS, NUM_EXPERTS), jnp.float32),
        compiler_params=pltpu.CompilerParams(
            dimension_semantics=("parallel",),
        ),
    )(x, wt, b2)


# speedup vs baseline: 1.0471x; 1.0471x over previous
import jax
import jax.numpy as jnp
from jax.experimental import pallas as pl
from jax.experimental.pallas import tpu as pltpu

N_TOKENS = 16384
HIDDEN_DIM = 2048
NUM_EXPERTS = 64
TILE = 1024


def _router_kernel(x_ref, w_ref, b_ref, o_ref):
    x = x_ref[...].astype(jnp.bfloat16)
    w = w_ref[...].astype(jnp.bfloat16)
    logits = jax.lax.dot_general(
        x, w, (((1,), (1,)), ((), ())),
        preferred_element_type=jnp.float32,
    ) + b_ref[...]
    m = jnp.max(logits, axis=-1, keepdims=True)
    e = jnp.exp(logits - m)
    o_ref[...] = e / jnp.sum(e, axis=-1, keepdims=True)


def kernel(x, W, b):
    b2 = b.reshape(1, NUM_EXPERTS)
    return pl.pallas_call(
        _router_kernel,
        grid=(N_TOKENS // TILE,),
        in_specs=[
            pl.BlockSpec((TILE, HIDDEN_DIM), lambda i: (i, 0)),
            pl.BlockSpec((NUM_EXPERTS, HIDDEN_DIM), lambda i: (0, 0)),
            pl.BlockSpec((1, NUM_EXPERTS), lambda i: (0, 0)),
        ],
        out_specs=pl.BlockSpec((TILE, NUM_EXPERTS), lambda i: (i, 0)),
        out_shape=jax.ShapeDtypeStruct((N_TOKENS, NUM_EXPERTS), jnp.float32),
        compiler_params=pltpu.CompilerParams(
            dimension_semantics=("parallel",),
        ),
    )(x, W, b2)
